# Initial kernel scaffold; baseline (speedup 1.0000x reference)
import functools

import jax
import jax.numpy as jnp
from jax import lax
from jax.experimental import pallas as pl
from jax.experimental.pallas import tpu as pltpu
from jax.experimental.pallas import tpu_sc as plsc

E = 320000
D = 128
L = 16
NUM_WORKERS = 32
E_PER_W = E // NUM_WORKERS
C = 200
NCHUNK = E_PER_W // C

_mesh = plsc.VectorSubcoreMesh(core_axis_name="c", subcore_axis_name="s")


@functools.partial(
    pl.kernel,
    mesh=_mesh,
    out_type=jax.ShapeDtypeStruct((E,), jnp.float32),
    scratch_types=[
        pltpu.VMEM((E_PER_W,), jnp.int32),
        pltpu.VMEM((E_PER_W,), jnp.int32),
        pltpu.VMEM((C, D), jnp.float32),
        pltpu.VMEM((C, D), jnp.float32),
        pltpu.VMEM((E_PER_W,), jnp.float32),
        pltpu.SemaphoreType.DMA,
        pltpu.SemaphoreType.DMA,
    ],
)
def _decode(z_hbm, src_hbm, dst_hbm, out_hbm,
            src_idx, dst_idx, srows, drows, outv, sem_s, sem_d):
    wid = lax.axis_index("s") * 2 + lax.axis_index("c")
    base = wid * E_PER_W

    pltpu.sync_copy(src_hbm.at[pl.ds(base, E_PER_W)], src_idx)
    pltpu.sync_copy(dst_hbm.at[pl.ds(base, E_PER_W)], dst_idx)

    def chunk_body(i, carry):
        off = i * C
        cp_s = pltpu.async_copy(z_hbm.at[src_idx.at[pl.ds(off, C)]], srows, sem_s)
        cp_d = pltpu.async_copy(z_hbm.at[dst_idx.at[pl.ds(off, C)]], drows, sem_d)
        cp_s.wait()
        cp_d.wait()

        def edge_body(e, carry2):
            acc = srows[e, pl.ds(0, L)] * drows[e, pl.ds(0, L)]
            for k in range(1, D // L):
                acc = acc + srows[e, pl.ds(k * L, L)] * drows[e, pl.ds(k * L, L)]
            outv[off + e] = jnp.sum(acc)
            return carry2

        lax.fori_loop(0, C, edge_body, 0, unroll=2)
        return carry

    lax.fori_loop(0, NCHUNK, chunk_body, 0)

    def sig_body(j, carry):
        v = outv[pl.ds(j * L, L)]
        outv[pl.ds(j * L, L)] = 1.0 / (1.0 + jnp.exp(-v))
        return carry

    lax.fori_loop(0, E_PER_W // L, sig_body, 0)

    pltpu.sync_copy(outv, out_hbm.at[pl.ds(base, E_PER_W)])


def kernel(z, edge_index):
    idx = edge_index.astype(jnp.int32)
    return _decode(z, idx[0], idx[1])


# speedup vs baseline: 1.1786x; 1.1786x over previous
import functools

import jax
import jax.numpy as jnp
from jax import lax
from jax.experimental import pallas as pl
from jax.experimental.pallas import tpu as pltpu
from jax.experimental.pallas import tpu_sc as plsc

E = 320000
D = 128
L = 16
NUM_WORKERS = 32
E_PER_W = E // NUM_WORKERS
C = 80
NCHUNK = E_PER_W // C
G = C // L

_mesh = plsc.VectorSubcoreMesh(core_axis_name="c", subcore_axis_name="s")


@functools.partial(
    pl.kernel,
    mesh=_mesh,
    out_type=jax.ShapeDtypeStruct((E,), jnp.float32),
    compiler_params=pltpu.CompilerParams(needs_layout_passes=False),
    scratch_types=[
        pltpu.VMEM((E_PER_W,), jnp.int32),
        pltpu.VMEM((E_PER_W,), jnp.int32),
        pltpu.VMEM((C, D), jnp.float32),
        pltpu.VMEM((C, D), jnp.float32),
        pltpu.VMEM((E_PER_W,), jnp.float32),
        pltpu.SemaphoreType.DMA,
        pltpu.SemaphoreType.DMA,
    ],
)
def _decode(z_hbm, src_hbm, dst_hbm, out_hbm,
            src_idx, dst_idx, srows, drows, outv, sem_s, sem_d):
    wid = lax.axis_index("s") * 2 + lax.axis_index("c")
    base = wid * E_PER_W

    pltpu.sync_copy(src_hbm.at[pl.ds(base, E_PER_W)], src_idx)
    pltpu.sync_copy(dst_hbm.at[pl.ds(base, E_PER_W)], dst_idx)

    def chunk_body(i, carry):
        off = i * C
        cp_s = pltpu.async_copy(z_hbm.at[src_idx.at[pl.ds(off, C)]], srows, sem_s)
        cp_d = pltpu.async_copy(z_hbm.at[dst_idx.at[pl.ds(off, C)]], drows, sem_d)
        cp_s.wait()
        cp_d.wait()

        def group_body(g, carry2):
            rows = g * L + lax.iota(jnp.int32, L)

            def col_body(c, acc):
                cols = jnp.full((L,), c, jnp.int32)
                sv = plsc.load_gather(srows, [rows, cols])
                dv = plsc.load_gather(drows, [rows, cols])
                return acc + sv * dv

            dots = lax.fori_loop(0, D, col_body, jnp.zeros((L,), jnp.float32),
                                 unroll=8)
            outv[pl.ds(off + g * L, L)] = dots
            return carry2

        lax.fori_loop(0, G, group_body, 0)
        return carry

    lax.fori_loop(0, NCHUNK, chunk_body, 0)

    def sig_body(j, carry):
        v = outv[pl.ds(j * L, L)]
        outv[pl.ds(j * L, L)] = 1.0 / (1.0 + jnp.exp(-v))
        return carry

    lax.fori_loop(0, E_PER_W // L, sig_body, 0)

    pltpu.sync_copy(outv, out_hbm.at[pl.ds(base, E_PER_W)])


def kernel(z, edge_index):
    idx = edge_index.astype(jnp.int32)
    return _decode(z, idx[0], idx[1])


# speedup vs baseline: 3.9879x; 3.3836x over previous
import functools

import jax
import jax.numpy as jnp
from jax import lax
from jax.experimental import pallas as pl
from jax.experimental.pallas import tpu as pltpu
from jax.experimental.pallas import tpu_sc as plsc

E = 320000
D = 128
L = 16
NUM_WORKERS = 32
E_PER_W = E // NUM_WORKERS
C = 80
NCHUNK = E_PER_W // C
G = C // L

_mesh = plsc.VectorSubcoreMesh(core_axis_name="c", subcore_axis_name="s")


@functools.partial(
    pl.kernel,
    mesh=_mesh,
    out_type=jax.ShapeDtypeStruct((E,), jnp.float32),
    compiler_params=pltpu.CompilerParams(needs_layout_passes=False),
    scratch_types=[
        pltpu.VMEM((E_PER_W,), jnp.int32),
        pltpu.VMEM((E_PER_W,), jnp.int32),
        pltpu.VMEM((C, D), jnp.float32),
        pltpu.VMEM((C, D), jnp.float32),
        pltpu.VMEM((E_PER_W,), jnp.float32),
        pltpu.VMEM((C, L + 1), jnp.float32),
        pltpu.SemaphoreType.DMA,
        pltpu.SemaphoreType.DMA,
    ],
)
def _decode(z_hbm, src_hbm, dst_hbm, out_hbm,
            src_idx, dst_idx, srows, drows, outv, pad, sem_s, sem_d):
    wid = lax.axis_index("s") * 2 + lax.axis_index("c")
    base = wid * E_PER_W

    pltpu.sync_copy(src_hbm.at[pl.ds(base, E_PER_W)], src_idx)
    pltpu.sync_copy(dst_hbm.at[pl.ds(base, E_PER_W)], dst_idx)

    def chunk_body(i, carry):
        off = i * C
        cp_s = pltpu.async_copy(z_hbm.at[src_idx.at[pl.ds(off, C)]], srows, sem_s)
        cp_d = pltpu.async_copy(z_hbm.at[dst_idx.at[pl.ds(off, C)]], drows, sem_d)
        cp_s.wait()
        cp_d.wait()

        def edge_body(e, carry2):
            acc = srows[e, pl.ds(0, L)] * drows[e, pl.ds(0, L)]
            for k in range(1, D // L):
                acc = acc + srows[e, pl.ds(k * L, L)] * drows[e, pl.ds(k * L, L)]
            pad[e, pl.ds(0, L)] = acc
            return carry2

        lax.fori_loop(0, C, edge_body, 0, unroll=2)

        lanes = lax.iota(jnp.int32, L)

        def group_body(g, carry2):
            rows = g * L + lanes
            dots = plsc.load_gather(pad, [rows, jnp.zeros((L,), jnp.int32)])
            for j in range(1, L):
                dots = dots + plsc.load_gather(pad, [rows, jnp.full((L,), j, jnp.int32)])
            outv[pl.ds(off + g * L, L)] = dots
            return carry2

        lax.fori_loop(0, G, group_body, 0)
        return carry

    lax.fori_loop(0, NCHUNK, chunk_body, 0)

    def sig_body(j, carry):
        v = outv[pl.ds(j * L, L)]
        outv[pl.ds(j * L, L)] = 1.0 / (1.0 + jnp.exp(-v))
        return carry

    lax.fori_loop(0, E_PER_W // L, sig_body, 0)

    pltpu.sync_copy(outv, out_hbm.at[pl.ds(base, E_PER_W)])


def kernel(z, edge_index):
    idx = edge_index.astype(jnp.int32)
    return _decode(z, idx[0], idx[1])


# speedup vs baseline: 8.0200x; 2.0111x over previous
import functools

import jax
import jax.numpy as jnp
from jax import lax
from jax.experimental import pallas as pl
from jax.experimental.pallas import tpu as pltpu
from jax.experimental.pallas import tpu_sc as plsc

E = 320000
D = 128
L = 16
NUM_WORKERS = 32
E_PER_W = E // NUM_WORKERS
C = 80
NCHUNK = E_PER_W // C
G = C // L
PADW = L + 1

_mesh = plsc.VectorSubcoreMesh(core_axis_name="c", subcore_axis_name="s")


@functools.partial(
    pl.kernel,
    mesh=_mesh,
    out_type=jax.ShapeDtypeStruct((E,), jnp.float32),
    compiler_params=pltpu.CompilerParams(needs_layout_passes=False),
    scratch_types=[
        pltpu.VMEM((E_PER_W,), jnp.int32),
        pltpu.VMEM((E_PER_W,), jnp.int32),
        pltpu.VMEM((2, C, D), jnp.float32),
        pltpu.VMEM((2, C, D), jnp.float32),
        pltpu.VMEM((E_PER_W,), jnp.float32),
        pltpu.VMEM((C * PADW,), jnp.float32),
        pltpu.SemaphoreType.DMA,
        pltpu.SemaphoreType.DMA,
    ],
)
def _decode(z_hbm, src_hbm, dst_hbm, out_hbm,
            src_idx, dst_idx, srows, drows, outv, pad, sem_s, sem_d):
    wid = lax.axis_index("s") * 2 + lax.axis_index("c")
    base = wid * E_PER_W

    pltpu.sync_copy(src_hbm.at[pl.ds(base, E_PER_W)], src_idx)
    pltpu.sync_copy(dst_hbm.at[pl.ds(base, E_PER_W)], dst_idx)

    def issue(c, slot):
        off = c * C
        pltpu.async_copy(z_hbm.at[src_idx.at[pl.ds(off, C)]], srows.at[slot], sem_s)
        pltpu.async_copy(z_hbm.at[dst_idx.at[pl.ds(off, C)]], drows.at[slot], sem_d)

    def drain(c, slot):
        off = c * C
        pltpu.make_async_copy(
            z_hbm.at[src_idx.at[pl.ds(off, C)]], srows.at[slot], sem_s).wait()
        pltpu.make_async_copy(
            z_hbm.at[dst_idx.at[pl.ds(off, C)]], drows.at[slot], sem_d).wait()

    lanes = lax.iota(jnp.int32, L)

    def compute(c, slot):
        off = c * C
        sr = srows.at[slot]
        dr = drows.at[slot]

        def edge_body(e, carry):
            acc = sr[e, pl.ds(0, L)] * dr[e, pl.ds(0, L)]
            for k in range(1, D // L):
                acc = acc + sr[e, pl.ds(k * L, L)] * dr[e, pl.ds(k * L, L)]
            pad[pl.ds(e * PADW, L)] = acc
            return carry

        lax.fori_loop(0, C, edge_body, 0, unroll=2)

        def group_body(g, carry):
            rows = (g * L + lanes) * PADW
            dots = plsc.load_gather(pad, [rows])
            for j in range(1, L):
                dots = dots + plsc.load_gather(pad, [rows + j])
            outv[pl.ds(off + g * L, L)] = 1.0 / (1.0 + jnp.exp(-dots))
            return carry

        lax.fori_loop(0, G, group_body, 0)

    issue(0, 0)
    issue(1, 1)

    def step(s, carry):
        c0 = 2 * s
        drain(c0, 0)
        compute(c0, 0)
        issue(c0 + 2, 0)
        drain(c0 + 1, 1)
        compute(c0 + 1, 1)

        @pl.when(s < (NCHUNK - 3) // 2)
        def _():
            issue(c0 + 3, 1)

        return carry

    lax.fori_loop(0, (NCHUNK - 1) // 2, step, 0)
    drain(NCHUNK - 1, 0)
    compute(NCHUNK - 1, 0)

    pltpu.sync_copy(outv, out_hbm.at[pl.ds(base, E_PER_W)])


def kernel(z, edge_index):
    idx = edge_index.astype(jnp.int32)
    return _decode(z, idx[0], idx[1])
